# Initial kernel scaffold; baseline (speedup 1.0000x reference)
import jax
import jax.numpy as jnp
from jax.experimental import pallas as pl


def kernel(x, edge_index, batch, W_in, b_in, Wc0, bc0, g0, be0, Wc1, bc1, g1, be1, R1, rb1, Wc2, bc2, g2, be2, R2, rb2, A1, ab1, A2, ab2, C1, cb1, C2, cb2, C3, cb3):
    raise NotImplementedError("write your pallas kernel here")


# speedup vs baseline: 7.7664x; 7.7664x over previous
import functools

import jax
import jax.numpy as jnp
from jax import lax
from jax.experimental import pallas as pl
from jax.experimental.pallas import tpu as pltpu
from jax.experimental.pallas import tpu_sc as plsc

N = 10000
E = 320000
D_IN = 128
H0, H1, H2 = 512, 256, 128
G = 16
CLS = 5

_NC = 2
_NS = 16
_B = 128
_SB = 40
_EA = 20480
_EB = 10240
_KA = _EA // _B
_KB = _EB // _B
_NPAD = 10240
_RPT = _NPAD // _NS
_DUMP = 10016

_BN = 1000
_NB = N // _BN


def _vsc_mesh():
    return plsc.VectorSubcoreMesh(core_axis_name="c", subcore_axis_name="s",
                                  num_cores=_NC, num_subcores=_NS)


def _make_msg(Q, KK, F):
    P = Q // _NC
    NSB = KK // _SB

    @functools.partial(
        pl.kernel,
        out_type=jax.ShapeDtypeStruct((Q, _NPAD, F), jnp.float32),
        mesh=_vsc_mesh(),
        scratch_types=[
            pltpu.VMEM((_SB * _B,), jnp.int32),
            pltpu.VMEM((_SB, _B), jnp.int32),
            pltpu.VMEM((_B, F), jnp.float32),
            pltpu.VMEM((_B, F), jnp.float32),
            pltpu.VMEM_SHARED((_NPAD, F), jnp.float32),
            pltpu.SemaphoreType.DMA,
            pltpu.SemaphoreType.DMA,
        ],
    )
    def msg(y_hbm, src_hbm, dst_hbm, out_hbm, src1d, dst2d, buf0, buf1, acc, sem0, sem1):
        cid = lax.axis_index("c")
        sid = lax.axis_index("s")
        row0 = sid * _RPT
        zeros16 = jnp.zeros((16,), jnp.float32)
        bufs = (buf0, buf1)
        sems = (sem0, sem1)

        def gath(j, buf, sem):
            pltpu.async_copy(y_hbm.at[src1d.at[pl.ds(j * _B, _B)]], buf, sem)

        def gwait(j, buf, sem):
            pltpu.make_async_copy(y_hbm.at[src1d.at[pl.ds(j * _B, _B)]], buf, sem).wait()

        for p in range(P):
            q = cid * P + p

            def zrow(r, carry):
                for f in range(F // 16):
                    buf0[r, pl.ds(f * 16, 16)] = zeros16
                return carry

            lax.fori_loop(0, _B, zrow, 0)
            for z in range(_RPT // _B):
                pltpu.sync_copy(buf0, acc.at[pl.ds(row0 + z * _B, _B)])
            plsc.subcore_barrier()

            def sb_loop(sb, carry):
                pltpu.sync_copy(src_hbm.at[q, sid, pl.ds(sb * _SB * _B, _SB * _B)], src1d)
                pltpu.sync_copy(dst_hbm.at[q, sid, pl.ds(sb * _SB, _SB)], dst2d)

                def m_loop(m, carry2):
                    base = m * 8
                    gath(base, buf0, sem0)
                    for b in range(8):
                        if b < 7:
                            gath(base + b + 1, bufs[(b + 1) % 2], sems[(b + 1) % 2])
                        gwait(base + b, bufs[b % 2], sems[b % 2])
                        pltpu.sync_copy(bufs[b % 2], acc.at[dst2d.at[base + b]], add=True)
                    return carry2

                lax.fori_loop(0, _SB // 8, m_loop, 0)
                return carry

            lax.fori_loop(0, NSB, sb_loop, 0)
            plsc.subcore_barrier()
            pltpu.sync_copy(acc.at[pl.ds(row0, _RPT)], out_hbm.at[q, pl.ds(row0, _RPT)])

    return msg


def _make_deg():
    @functools.partial(
        pl.kernel,
        out_type=jax.ShapeDtypeStruct((_NC, _NPAD, 16), jnp.float32),
        mesh=_vsc_mesh(),
        scratch_types=[
            pltpu.VMEM((_KB, _B), jnp.int32),
            pltpu.VMEM((_B, 16), jnp.float32),
            pltpu.VMEM_SHARED((_NPAD, 16), jnp.float32),
        ],
    )
    def deg(dst_hbm, out_hbm, dst_v, ones_v, acc):
        cid = lax.axis_index("c")
        sid = lax.axis_index("s")
        zeros16 = jnp.zeros((16,), jnp.float32)
        ones16 = jnp.ones((16,), jnp.float32)

        def zrow(r, carry):
            ones_v[r] = zeros16
            return carry

        lax.fori_loop(0, _B, zrow, 0)
        row0 = sid * _RPT
        for z in range(_RPT // _B):
            pltpu.sync_copy(ones_v, acc.at[pl.ds(row0 + z * _B, _B)])

        def orow(r, carry):
            ones_v[r] = ones16
            return carry

        lax.fori_loop(0, _B, orow, 0)
        pltpu.sync_copy(dst_hbm.at[cid, sid], dst_v)
        plsc.subcore_barrier()

        def step(j, carry):
            pltpu.sync_copy(ones_v, acc.at[dst_v.at[j]], add=True)
            return carry

        lax.fori_loop(0, _KB, step, 0)
        plsc.subcore_barrier()
        pltpu.sync_copy(acc.at[pl.ds(row0, _RPT)], out_hbm.at[cid, pl.ds(row0, _RPT)])

    return deg


def _k_in_body(x_ref, w_ref, b_ref, d0_ref, d1_ref, h_ref, dinv_ref):
    h = jnp.dot(x_ref[...], w_ref[...], preferred_element_type=jnp.float32)
    h_ref[...] = jnp.maximum(h + b_ref[...][None, :], 0.0)
    deg = d0_ref[:, 0:1] + d1_ref[:, 0:1] + 1.0
    dinv_ref[...] = lax.rsqrt(deg)


def _make_k_in(interpret=False):
    return pl.pallas_call(
        _k_in_body,
        grid=(_NB,),
        in_specs=[
            pl.BlockSpec((_BN, D_IN), lambda i: (i, 0)),
            pl.BlockSpec((D_IN, H0), lambda i: (0, 0)),
            pl.BlockSpec((H0,), lambda i: (0,)),
            pl.BlockSpec((_BN, 16), lambda i: (i, 0)),
            pl.BlockSpec((_BN, 16), lambda i: (i, 0)),
        ],
        out_specs=[
            pl.BlockSpec((_BN, H0), lambda i: (i, 0)),
            pl.BlockSpec((_BN, 1), lambda i: (i, 0)),
        ],
        out_shape=[
            jax.ShapeDtypeStruct((N, H0), jnp.float32),
            jax.ShapeDtypeStruct((N, 1), jnp.float32),
        ],
        interpret=interpret,
    )


def _k_pre0_body(h_ref, w_ref, dinv_ref, y_ref):
    y_ref[0] = jnp.dot(h_ref[...], w_ref[...], preferred_element_type=jnp.float32) * dinv_ref[...]


def _make_k_pre0(C, F, Hin, interpret=False):
    return pl.pallas_call(
        _k_pre0_body,
        grid=(_NB, C),
        in_specs=[
            pl.BlockSpec((_BN, Hin), lambda i, c: (i, 0)),
            pl.BlockSpec((Hin, F), lambda i, c: (0, c)),
            pl.BlockSpec((_BN, 1), lambda i, c: (i, 0)),
        ],
        out_specs=[pl.BlockSpec((1, _BN, F), lambda i, c: (c, i, 0))],
        out_shape=[jax.ShapeDtypeStruct((C, N, F), jnp.float32)],
        interpret=interpret,
    )


def _k_pre_body(h_ref, w_ref, r_ref, rb_ref, dinv_ref, y_ref, res_ref):
    hb = h_ref[...]
    y_ref[0] = jnp.dot(hb, w_ref[...], preferred_element_type=jnp.float32) * dinv_ref[...]
    res_ref[...] = jnp.dot(hb, r_ref[...], preferred_element_type=jnp.float32) + rb_ref[...][None, :]


def _make_k_pre(C, F, Hin, interpret=False):
    Hout = C * F
    return pl.pallas_call(
        _k_pre_body,
        grid=(_NB, C),
        in_specs=[
            pl.BlockSpec((_BN, Hin), lambda i, c: (i, 0)),
            pl.BlockSpec((Hin, F), lambda i, c: (0, c)),
            pl.BlockSpec((Hin, F), lambda i, c: (0, c)),
            pl.BlockSpec((F,), lambda i, c: (c,)),
            pl.BlockSpec((_BN, 1), lambda i, c: (i, 0)),
        ],
        out_specs=[
            pl.BlockSpec((1, _BN, F), lambda i, c: (c, i, 0)),
            pl.BlockSpec((_BN, F), lambda i, c: (i, c)),
        ],
        out_shape=[
            jax.ShapeDtypeStruct((C, N, F), jnp.float32),
            jax.ShapeDtypeStruct((N, Hout), jnp.float32),
        ],
        interpret=interpret,
    )


def _make_k_post_body(C, partials):
    def body(t_ref, y_ref, dinv_ref, b_ref, g_ref, be_ref, res_ref, out_ref):
        if partials:
            s = t_ref[0] + t_ref[1] + y_ref[...]
        else:
            parts = [t_ref[c] + y_ref[c] for c in range(C)]
            s = jnp.concatenate(parts, axis=-1) if C > 1 else parts[0]
        s = s * dinv_ref[...] + b_ref[...][None, :]
        mu = jnp.mean(s, axis=-1, keepdims=True)
        var = jnp.mean((s - mu) ** 2, axis=-1, keepdims=True)
        sn = (s - mu) * lax.rsqrt(var + 1e-5) * g_ref[...][None, :] + be_ref[...][None, :]
        out_ref[...] = jnp.maximum(sn, 0.0) + res_ref[...]

    return body


def _make_k_post(C, F, partials=False, interpret=False):
    Hout = C * F
    nt = _NC if partials else C
    return pl.pallas_call(
        _make_k_post_body(C, partials),
        grid=(_NB,),
        in_specs=[
            pl.BlockSpec((nt, _BN, F), lambda i: (0, i, 0)),
            pl.BlockSpec((_BN, Hout) if partials else (C, _BN, F),
                         (lambda i: (i, 0)) if partials else (lambda i: (0, i, 0))),
            pl.BlockSpec((_BN, 1), lambda i: (i, 0)),
            pl.BlockSpec((Hout,), lambda i: (0,)),
            pl.BlockSpec((Hout,), lambda i: (0,)),
            pl.BlockSpec((Hout,), lambda i: (0,)),
            pl.BlockSpec((_BN, Hout), lambda i: (i, 0)),
        ],
        out_specs=[pl.BlockSpec((_BN, Hout), lambda i: (i, 0))],
        out_shape=[jax.ShapeDtypeStruct((N, Hout), jnp.float32)],
        interpret=interpret,
    )


def _k_att_body(h_ref, a1_ref, ab1_ref, a2_ref, ab2_ref, batch_ref, out_ref):
    i = pl.program_id(0)
    h = h_ref[...]
    t = jnp.maximum(jnp.dot(h, a1_ref[...], preferred_element_type=jnp.float32)
                    + ab1_ref[...][None, :], 0.0)
    s = jnp.sum(t * a2_ref[...], axis=-1, keepdims=True) + ab2_ref[0]
    w = jax.nn.sigmoid(s)
    u = h * w
    b = batch_ref[0]
    labels = lax.broadcasted_iota(jnp.int32, (G, 1), 0)
    m = (b == labels).astype(jnp.float32)
    contrib = jnp.dot(m, u, preferred_element_type=jnp.float32)

    @pl.when(i == 0)
    def _():
        out_ref[...] = jnp.zeros_like(out_ref)

    out_ref[...] += contrib


def _make_k_att(interpret=False):
    return pl.pallas_call(
        _k_att_body,
        grid=(_NB,),
        in_specs=[
            pl.BlockSpec((_BN, H2), lambda i: (i, 0)),
            pl.BlockSpec((H2, H2 // 2), lambda i: (0, 0)),
            pl.BlockSpec((H2 // 2,), lambda i: (0,)),
            pl.BlockSpec((1, H2 // 2), lambda i: (0, 0)),
            pl.BlockSpec((1,), lambda i: (0,)),
            pl.BlockSpec((1, 1, _BN), lambda i: (i, 0, 0)),
        ],
        out_specs=[pl.BlockSpec((G, H2), lambda i: (0, 0))],
        out_shape=[jax.ShapeDtypeStruct((G, H2), jnp.float32)],
        interpret=interpret,
    )


def _k_mlp_body(p_ref, c1_ref, cb1_ref, c2_ref, cb2_ref, c3_ref, cb3_ref, out_ref):
    z = jnp.maximum(jnp.dot(p_ref[...], c1_ref[...], preferred_element_type=jnp.float32)
                    + cb1_ref[...][None, :], 0.0)
    z = jnp.maximum(jnp.dot(z, c2_ref[...], preferred_element_type=jnp.float32)
                    + cb2_ref[...][None, :], 0.0)
    out_ref[...] = jnp.dot(z, c3_ref[...], preferred_element_type=jnp.float32) + cb3_ref[...][None, :]


def _make_k_mlp(interpret=False):
    return pl.pallas_call(
        _k_mlp_body,
        in_specs=[
            pl.BlockSpec((G, H2), lambda: (0, 0)),
            pl.BlockSpec((H2, H2 // 2), lambda: (0, 0)),
            pl.BlockSpec((H2 // 2,), lambda: (0,)),
            pl.BlockSpec((H2 // 2, H2 // 4), lambda: (0, 0)),
            pl.BlockSpec((H2 // 4,), lambda: (0,)),
            pl.BlockSpec((H2 // 4, 128), lambda: (0, 0)),
            pl.BlockSpec((128,), lambda: (0,)),
        ],
        out_specs=[pl.BlockSpec((G, 128), lambda: (0, 0))],
        out_shape=[jax.ShapeDtypeStruct((G, 128), jnp.float32)],
        interpret=interpret,
    )


_get_msg = functools.cache(_make_msg)
_get_deg = functools.cache(_make_deg)
_k_in = _make_k_in()
_k_pre0 = _make_k_pre0(4, 128, H0)
_k_pre1 = _make_k_pre(2, 128, H0)
_k_pre2 = _make_k_pre(1, 128, H1)
_k_post0 = _make_k_post(4, 128)
_k_post1 = _make_k_post(2, 128)
_k_post2 = _make_k_post(1, 128, partials=True)
_k_att = _make_k_att()
_k_mlp = _make_k_mlp()


def kernel(x, edge_index, batch, W_in, b_in, Wc0, bc0, g0, be0, Wc1, bc1, g1, be1,
           R1, rb1, Wc2, bc2, g2, be2, R2, rb2, A1, ab1, A2, ab2, C1, cb1, C2, cb2,
           C3, cb3):
    src = edge_index[0].astype(jnp.int32)
    dst = edge_index[1].astype(jnp.int32)

    def prep(idx, nt, padded, fill):
        a = idx.reshape(nt, E // nt)
        return jnp.pad(a, ((0, 0), (0, padded - E // nt)), constant_values=fill)

    srcA = prep(src, _NS, _EA, 0)
    dstA = prep(dst, _NS, _EA, _DUMP).reshape(_NS, _KA, _B)
    srcB = prep(src, _NC * _NS, _EB, 0).reshape(_NC, _NS, _EB)
    dstB = prep(dst, _NC * _NS, _EB, _DUMP).reshape(_NC, _NS, _KB, _B)
    offs4 = (jnp.arange(4, dtype=jnp.int32) * N)[:, None, None]
    src4 = srcA[None] + offs4
    src2 = srcA[None] + offs4[:2]
    dst4 = jnp.broadcast_to(dstA[None], (4, _NS, _KA, _B))
    dst2 = jnp.broadcast_to(dstA[None], (2, _NS, _KA, _B))

    degp = _get_deg()(dstB)
    h0, dinv = _k_in(x, W_in, b_in, degp[0], degp[1])

    (y0,) = _k_pre0(h0, Wc0, dinv)
    t0 = _get_msg(4, _KA, 128)(y0.reshape(4 * N, 128), src4, dst4)
    (h1,) = _k_post0(t0, y0, dinv, bc0, g0, be0, h0)

    y1, r1 = _k_pre1(h1, Wc1, R1, rb1, dinv)
    t1 = _get_msg(2, _KA, 128)(y1.reshape(2 * N, 128), src2, dst2)
    (h2,) = _k_post1(t1, y1, dinv, bc1, g1, be1, r1)

    y2, r2 = _k_pre2(h2, Wc2, R2, rb2, dinv)
    y2f = y2.reshape(N, 128)
    t2 = _get_msg(2, _KB, 128)(y2f, srcB, dstB)
    (h3,) = _k_post2(t2, y2f, dinv, bc2, g2, be2, r2)

    batch3 = batch.astype(jnp.int32).reshape(_NB, 1, _BN)
    (pooled,) = _k_att(h3, A1, ab1, A2.reshape(1, H2 // 2), ab2, batch3)
    c3p = jnp.pad(C3, ((0, 0), (0, 128 - CLS)))
    cb3p = jnp.pad(cb3, (0, 128 - CLS))
    (outp,) = _k_mlp(pooled, C1, cb1, C2, cb2, c3p, cb3p)
    return outp[:, :CLS]
